# baseline (speedup 1.0000x reference)
import jax
import jax.numpy as jnp
from jax.experimental import pallas as pl
from jax.experimental.pallas import tpu as pltpu

HIDDEN = 256
LANE = 128
N_GCONV = 10
LEAKY_SLOPE = 0.01
MPS = 4


def _leaky_relu(x):
    return jnp.maximum(x, LEAKY_SLOPE * x)


def _body(x_ref, adj_ref,
          m1w0, m1b0, m1w1, m1b1, m1w2, m1b2, m1w3, m1b3,
          g0wh, g0wx, g0b, gw, gb,
          m3w0, m3b0, m3w1, m3b1, m3w2, m3b2, m3w3, m3b3,
          o_ref):
    f32 = jnp.float32
    bf16 = jnp.bfloat16
    V = x_ref.shape[1]
    x = x_ref[...].reshape(MPS * V, LANE)
    adjs = [adj_ref[m] for m in range(MPS)]

    def aggregate(h1):
        return jnp.concatenate(
            [jnp.dot(adjs[m], h1[m * V:(m + 1) * V],
                     preferred_element_type=f32) for m in range(MPS)], axis=0)

    h = x
    for w, b in ((m1w0, m1b0), (m1w1, m1b1), (m1w2, m1b2), (m1w3, m1b3)):
        h = _leaky_relu(
            jnp.dot(h, w[...], preferred_element_type=f32) + b[...]
        ).astype(bf16)

    hx = (jnp.dot(h, g0wh[...], preferred_element_type=f32)
          + jnp.dot(x, g0wx[...], preferred_element_type=f32) + g0b[...])
    h = jnp.maximum(
        hx[:, :HIDDEN] + aggregate(hx[:, HIDDEN:].astype(bf16)), 0.0
    ).astype(bf16)

    for i in range(N_GCONV - 1):
        hx = jnp.dot(h, gw[i], preferred_element_type=f32) + gb[i]
        h = jnp.maximum(
            hx[:, :HIDDEN] + aggregate(hx[:, HIDDEN:].astype(bf16)), 0.0
        ).astype(bf16)

    layers = ((m3w0, m3b0), (m3w1, m3b1), (m3w2, m3b2), (m3w3, m3b3))
    for i, (w, b) in enumerate(layers):
        o = jnp.dot(h, w[...], preferred_element_type=f32) + b[...]
        if i < len(layers) - 1:
            h = _leaky_relu(o).astype(bf16)
    o_ref[...] = o.reshape(MPS, V, LANE)


def kernel(verts, adj, p0, p1, p2, p3, p4, p5, p6, p7, p8, p9,
           p10, p11, p12, p13, p14, p15, p16, p17, p18, p19):
    B, V, _ = verts.shape
    bf16 = jnp.bfloat16
    x_pad = jnp.pad(verts, ((0, 0), (0, 0), (0, LANE - 3))).astype(bf16)
    adj = adj.astype(bf16)

    g0wh = jnp.concatenate([p8[0, :HIDDEN], p8[1, :HIDDEN]], axis=1).astype(bf16)
    g0wx = jnp.concatenate([p8[0, HIDDEN:], p8[1, HIDDEN:]], axis=1).astype(bf16)
    g0b = jnp.concatenate([p9[0], p9[1]], axis=1)
    gw = jnp.concatenate([p10[:, 0], p10[:, 1]], axis=-1).astype(bf16)
    gb = jnp.concatenate([p11[:, 0], p11[:, 1]], axis=-1)

    params = [p0.astype(bf16), p1, p2.astype(bf16), p3,
              p4.astype(bf16), p5, p6.astype(bf16), p7,
              g0wh, g0wx, g0b, gw, gb,
              p12.astype(bf16), p13, p14.astype(bf16), p15,
              p16.astype(bf16), p17, p18.astype(bf16), p19]

    def resident(a):
        zeros = (0,) * a.ndim
        return pl.BlockSpec(a.shape, lambda *_: zeros)

    in_specs = ([pl.BlockSpec((MPS, V, LANE), lambda b: (b, 0, 0)),
                 pl.BlockSpec((MPS, V, V), lambda b: (b, 0, 0))]
                + [resident(a) for a in params])

    out = pl.pallas_call(
        _body,
        out_shape=jax.ShapeDtypeStruct((B, V, LANE), jnp.float32),
        grid=(B // MPS,),
        in_specs=in_specs,
        out_specs=pl.BlockSpec((MPS, V, LANE), lambda b: (b, 0, 0)),
        compiler_params=pltpu.CompilerParams(
            dimension_semantics=("parallel",),
            vmem_limit_bytes=48 * 1024 * 1024),
    )(x_pad, adj, *params)

    return out[:, :, :3].reshape(B * V, 3)


# speedup vs baseline: 1.8961x; 1.8961x over previous
import jax
import jax.numpy as jnp
from jax.experimental import pallas as pl
from jax.experimental.pallas import tpu as pltpu

HIDDEN = 256
LANE = 128
N_GCONV = 10
LEAKY_SLOPE = 0.01
MPS = 2


def _leaky_relu(x):
    return jnp.maximum(x, LEAKY_SLOPE * x)


def _body(x_ref, adj_ref,
          m1w0, m1b0, m1w1, m1b1, m1w2, m1b2, m1w3, m1b3,
          g0wh, g0wx, g0b, gw, gb,
          m3w0, m3b0, m3w1, m3b1, m3w2, m3b2, m3w3, m3b3,
          o_ref):
    f32 = jnp.float32
    bf16 = jnp.bfloat16
    V = x_ref.shape[1]
    xs = [x_ref[m] for m in range(MPS)]
    adjs = [adj_ref[m].astype(bf16) for m in range(MPS)]


    hs = list(xs)
    for w, b in ((m1w0, m1b0), (m1w1, m1b1), (m1w2, m1b2), (m1w3, m1b3)):
        for m in range(MPS):
            hs[m] = _leaky_relu(
                jnp.dot(hs[m], w[...], preferred_element_type=f32) + b[...]
            ).astype(bf16)

    for m in range(MPS):
        hx = (jnp.dot(hs[m], g0wh[...], preferred_element_type=f32)
              + jnp.dot(xs[m], g0wx[...], preferred_element_type=f32)
              + g0b[...])
        agg = jnp.dot(adjs[m], hx[:, HIDDEN:].astype(bf16),
                      preferred_element_type=f32)
        hs[m] = jnp.maximum(hx[:, :HIDDEN] + agg, 0.0).astype(bf16)

    for i in range(N_GCONV - 1):
        for m in range(MPS):
            hx = jnp.dot(hs[m], gw[i], preferred_element_type=f32) + gb[i]
            agg = jnp.dot(adjs[m], hx[:, HIDDEN:].astype(bf16),
                          preferred_element_type=f32)
            hs[m] = jnp.maximum(hx[:, :HIDDEN] + agg, 0.0).astype(bf16)

    layers = ((m3w0, m3b0), (m3w1, m3b1), (m3w2, m3b2), (m3w3, m3b3))
    os = [None] * MPS
    for i, (w, b) in enumerate(layers):
        for m in range(MPS):
            os[m] = jnp.dot(hs[m], w[...], preferred_element_type=f32) + b[...]
            if i < len(layers) - 1:
                hs[m] = _leaky_relu(os[m]).astype(bf16)
    for m in range(MPS):
        o_ref[m] = os[m]


def kernel(verts, adj, p0, p1, p2, p3, p4, p5, p6, p7, p8, p9,
           p10, p11, p12, p13, p14, p15, p16, p17, p18, p19):
    B, V, _ = verts.shape
    bf16 = jnp.bfloat16
    x_pad = jnp.pad(verts, ((0, 0), (0, 0), (0, LANE - 3))).astype(bf16)

    g0wh = jnp.concatenate([p8[0, :HIDDEN], p8[1, :HIDDEN]], axis=1).astype(bf16)
    g0wx = jnp.concatenate([p8[0, HIDDEN:], p8[1, HIDDEN:]], axis=1).astype(bf16)
    g0b = jnp.concatenate([p9[0], p9[1]], axis=1)
    gw = jnp.concatenate([p10[:, 0], p10[:, 1]], axis=-1).astype(bf16)
    gb = jnp.concatenate([p11[:, 0], p11[:, 1]], axis=-1)

    params = [p0.astype(bf16), p1, p2.astype(bf16), p3,
              p4.astype(bf16), p5, p6.astype(bf16), p7,
              g0wh, g0wx, g0b, gw, gb,
              p12.astype(bf16), p13, p14.astype(bf16), p15,
              p16.astype(bf16), p17, p18.astype(bf16), p19]

    def resident(a):
        zeros = (0,) * a.ndim
        return pl.BlockSpec(a.shape, lambda *_: zeros)

    in_specs = ([pl.BlockSpec((MPS, V, LANE), lambda b: (b, 0, 0)),
                 pl.BlockSpec((MPS, V, V), lambda b: (b, 0, 0))]
                + [resident(a) for a in params])

    out = pl.pallas_call(
        _body,
        out_shape=jax.ShapeDtypeStruct((B, V, LANE), jnp.float32),
        grid=(B // MPS,),
        in_specs=in_specs,
        out_specs=pl.BlockSpec((MPS, V, LANE), lambda b: (b, 0, 0)),
        compiler_params=pltpu.CompilerParams(
            dimension_semantics=("parallel",),
            vmem_limit_bytes=48 * 1024 * 1024),
    )(x_pad, adj, *params)

    return out[:, :, :3].reshape(B * V, 3)
